# baseline (speedup 1.0000x reference)
import functools

import jax
import jax.numpy as jnp
from jax.experimental import pallas as pl
from jax.experimental.pallas import tpu as pltpu

ND = 1024
NM = 3072
N = ND + NM
E = 32768
HID = 64

_HI = jax.lax.Precision.HIGHEST


def _mm_body(a_ref, b_ref, o_ref):
    o_ref[...] = jnp.dot(a_ref[...], b_ref[...],
                         preferred_element_type=jnp.float32, precision=_HI)


def _matmul(a, b, bm=256):
    M, K = a.shape
    _, Nn = b.shape
    return pl.pallas_call(
        _mm_body,
        grid=(M // bm,),
        in_specs=[pl.BlockSpec((bm, K), lambda i: (i, 0)),
                  pl.BlockSpec((K, Nn), lambda i: (0, 0))],
        out_specs=pl.BlockSpec((bm, Nn), lambda i: (i, 0)),
        out_shape=jax.ShapeDtypeStruct((M, Nn), jnp.float32),
    )(a, b)


def _d2_body(g_ref, gt_ref, sq_ref, sqall_ref, o_ref):
    dot = jnp.dot(g_ref[...], gt_ref[...],
                  preferred_element_type=jnp.float32, precision=_HI)
    o_ref[...] = sq_ref[...].reshape(-1, 1) + sqall_ref[...] - 2.0 * dot


def _pairwise_d2(g, sq, bm=512):
    gt = g.T
    sq2 = sq.reshape(N, 1)
    sqr = sq.reshape(1, N)
    return pl.pallas_call(
        _d2_body,
        grid=(N // bm,),
        in_specs=[pl.BlockSpec((bm, g.shape[1]), lambda i: (i, 0)),
                  pl.BlockSpec((g.shape[1], N), lambda i: (0, 0)),
                  pl.BlockSpec((bm, 1), lambda i: (i, 0)),
                  pl.BlockSpec((1, N), lambda i: (0, 0))],
        out_specs=pl.BlockSpec((bm, N), lambda i: (i, 0)),
        out_shape=jax.ShapeDtypeStruct((N, N), jnp.float32),
    )(g, gt, sq2, sqr)


def _gat_edges(x, src, dst, W, a_s, a_d, b, heads, ch):
    h = _matmul(x, W).reshape(N, heads, ch)
    es = jnp.sum(h * a_s[None], axis=-1)
    ed = jnp.sum(h * a_d[None], axis=-1)
    e = jax.nn.leaky_relu(es[src] + ed[dst], 0.2)
    emax = jax.ops.segment_max(e, dst, num_segments=N)
    emax = jnp.where(jnp.isfinite(emax), emax, 0.0)
    ex = jnp.exp(e - emax[dst])
    den = jax.ops.segment_sum(ex, dst, num_segments=N)
    alpha = ex / (den[dst] + 1e-16)
    out = jax.ops.segment_sum(h[src] * alpha[:, :, None], dst, num_segments=N)
    return out.reshape(N, heads * ch) + b


def kernel(d_sim, m_sim, diseases, mirnas, W_d, W_m, W_lin1, b_lin1, W1, a_s1, a_d1, b1, W2, a_s2, a_d2, b2, Wp, bp, ln_g, ln_b, Wl, bl, Wd1, bd1, Wm1, bm1, Wpred, bpred):
    z_d = _matmul(d_sim, W_d)
    z_m = _matmul(m_sim, W_m)
    feats = jnp.concatenate([z_d, z_m], axis=0)
    src = diseases
    dst = mirnas

    x1 = jax.nn.relu(_matmul(feats, W_lin1) + b_lin1)
    xg = jax.nn.relu(_gat_edges(feats, src, dst, W1, a_s1, a_d1, b1, 8, 256))
    x2 = _gat_edges(xg, src, dst, W2, a_s2, a_d2, b2, 1, 256)
    final = jnp.concatenate([x1, x2], axis=1)

    Wp_pad = jnp.pad(Wp, ((0, 0), (0, 124)))
    bp_pad = jnp.pad(bp, (0, 124))
    g = _matmul(final, Wp_pad) + bp_pad
    sq = jnp.sum(g * g, axis=1)
    d2 = _pairwise_d2(g, sq)
    _, idx = jax.lax.top_k(-d2, 50)
    sim = jnp.mean(jnp.take(final, idx.reshape(-1), axis=0).reshape(N, 50, -1), axis=1)
    mu = jnp.mean(sim, axis=-1, keepdims=True)
    var = jnp.var(sim, axis=-1, keepdims=True)
    sim_n = (sim - mu) / jnp.sqrt(var + 1e-5) * ln_g + ln_b

    outf = _matmul(jnp.concatenate([final, sim_n], axis=1), Wl) + bl
    h_d = jnp.concatenate([outf[:ND], feats[:ND]], axis=1)
    h_m = jnp.concatenate([outf[ND:], feats[ND:]], axis=1)
    h_d = jax.nn.elu(_matmul(h_d, Wd1) + bd1)
    h_m = jax.nn.elu(_matmul(h_m, Wm1) + bm1)
    h = jnp.concatenate([h_d, h_m], axis=0)
    hc = jnp.concatenate([h[diseases], h[mirnas]], axis=1)
    Wpred_pad = jnp.pad(Wpred, ((0, 0), (0, 127)))
    logits = _matmul(hc, Wpred_pad)[:, :1]
    return jax.nn.sigmoid(logits + bpred)


# speedup vs baseline: 2.2302x; 2.2302x over previous
import functools

import jax
import jax.numpy as jnp
from jax.experimental import pallas as pl
from jax.experimental.pallas import tpu as pltpu

ND = 1024
NM = 3072
N = ND + NM
E = 32768
HID = 64

_HI = jax.lax.Precision.HIGHEST


def _mm_body(a_ref, b_ref, o_ref):
    o_ref[...] = jnp.dot(a_ref[...], b_ref[...],
                         preferred_element_type=jnp.float32, precision=_HI)


def _matmul(a, b, bm=256):
    M, K = a.shape
    _, Nn = b.shape
    return pl.pallas_call(
        _mm_body,
        grid=(M // bm,),
        in_specs=[pl.BlockSpec((bm, K), lambda i: (i, 0)),
                  pl.BlockSpec((K, Nn), lambda i: (0, 0))],
        out_specs=pl.BlockSpec((bm, Nn), lambda i: (i, 0)),
        out_shape=jax.ShapeDtypeStruct((M, Nn), jnp.float32),
    )(a, b)


_K = 50


def _f32_sortkey(x):
    b = jax.lax.bitcast_convert_type(x, jnp.int32)
    imin = jnp.full(b.shape, -(2 ** 31), jnp.int32)
    return jnp.where(b >= 0, b, jnp.bitwise_xor(jnp.bitwise_not(b), imin))


def _knn_sel_body(g_ref, gt_ref, key_ref, v50_ref, tie_ref):
    gblk = g_ref[...]
    gt = gt_ref[...]
    dot = jnp.dot(gblk, gt, preferred_element_type=jnp.float32, precision=_HI)
    sq_row = jnp.sum(gblk * gblk, axis=1, keepdims=True)
    sq_col = jnp.sum(gt * gt, axis=0, keepdims=True)
    d2 = sq_row + sq_col - 2.0 * dot
    key = _f32_sortkey(d2)
    key_ref[...] = key
    lo = jnp.min(key, axis=1, keepdims=True)
    hi = jnp.max(key, axis=1, keepdims=True)

    def vstep(_, carry):
        lo, hi = carry
        mid = (lo >> 1) + (hi >> 1) + (lo & hi & 1)
        cnt = jnp.sum((key <= mid).astype(jnp.int32), axis=1, keepdims=True)
        sel = cnt >= _K
        return jnp.where(sel, lo, mid + 1), jnp.where(sel, mid, hi)

    lo, hi = jax.lax.fori_loop(0, 32, vstep, (lo, hi))
    v50 = lo
    v50_ref[...] = v50
    r = _K - jnp.sum((key < v50).astype(jnp.int32), axis=1, keepdims=True)
    eq = (key == v50)
    col = jax.lax.broadcasted_iota(jnp.int32, key.shape, 1)
    tlo = jnp.zeros_like(v50)
    thi = jnp.full_like(v50, key.shape[1] - 1)

    def tstep(_, carry):
        tlo, thi = carry
        tmid = (tlo + thi) >> 1
        cnt = jnp.sum((eq & (col <= tmid)).astype(jnp.int32), axis=1,
                      keepdims=True)
        sel = cnt >= r
        return jnp.where(sel, tlo, tmid + 1), jnp.where(sel, tmid, thi)

    tlo, thi = jax.lax.fori_loop(0, 12, tstep, (tlo, thi))
    tie_ref[...] = tlo


def _knn_select(g, bm=256):
    gt = g.T
    return pl.pallas_call(
        _knn_sel_body,
        grid=(N // bm,),
        in_specs=[pl.BlockSpec((bm, g.shape[1]), lambda i: (i, 0)),
                  pl.BlockSpec((g.shape[1], N), lambda i: (0, 0))],
        out_specs=[pl.BlockSpec((bm, N), lambda i: (i, 0)),
                   pl.BlockSpec((bm, 1), lambda i: (i, 0)),
                   pl.BlockSpec((bm, 1), lambda i: (i, 0))],
        out_shape=[jax.ShapeDtypeStruct((N, N), jnp.int32),
                   jax.ShapeDtypeStruct((N, 1), jnp.int32),
                   jax.ShapeDtypeStruct((N, 1), jnp.int32)],
    )(g, gt)


def _knn_mix_body(key_ref, v50_ref, tie_ref, fin_ref, finblk_ref, lng_ref,
                  lnb_ref, wla_ref, wlb_ref, bl_ref, o_ref):
    key = key_ref[...]
    v50 = v50_ref[...]
    col = jax.lax.broadcasted_iota(jnp.int32, key.shape, 1)
    mask = ((key < v50) | ((key == v50) & (col <= tie_ref[...]))).astype(
        jnp.float32)
    sim = jnp.dot(mask, fin_ref[...], preferred_element_type=jnp.float32,
                  precision=_HI) * (1.0 / _K)
    mu = jnp.mean(sim, axis=1, keepdims=True)
    var = jnp.mean((sim - mu) ** 2, axis=1, keepdims=True)
    sim_n = (sim - mu) / jnp.sqrt(var + 1e-5) * lng_ref[...] + lnb_ref[...]
    out = jnp.dot(finblk_ref[...], wla_ref[...],
                  preferred_element_type=jnp.float32, precision=_HI)
    out += jnp.dot(sim_n, wlb_ref[...], preferred_element_type=jnp.float32,
                   precision=_HI)
    o_ref[...] = out + bl_ref[...]


def _knn_mix(key, v50, tie, final, ln_g, ln_b, Wl, bl, bm=256):
    F = final.shape[1]
    out_dim = Wl.shape[1]
    return pl.pallas_call(
        _knn_mix_body,
        grid=(N // bm,),
        in_specs=[pl.BlockSpec((bm, N), lambda i: (i, 0)),
                  pl.BlockSpec((bm, 1), lambda i: (i, 0)),
                  pl.BlockSpec((bm, 1), lambda i: (i, 0)),
                  pl.BlockSpec((N, F), lambda i: (0, 0)),
                  pl.BlockSpec((bm, F), lambda i: (i, 0)),
                  pl.BlockSpec((1, F), lambda i: (0, 0)),
                  pl.BlockSpec((1, F), lambda i: (0, 0)),
                  pl.BlockSpec((F, out_dim), lambda i: (0, 0)),
                  pl.BlockSpec((F, out_dim), lambda i: (0, 0)),
                  pl.BlockSpec((1, out_dim), lambda i: (0, 0))],
        out_specs=pl.BlockSpec((bm, out_dim), lambda i: (i, 0)),
        out_shape=jax.ShapeDtypeStruct((N, out_dim), jnp.float32),
    )(key, v50, tie, final, final, ln_g.reshape(1, F), ln_b.reshape(1, F),
      Wl[:F], Wl[F:], bl.reshape(1, out_dim))


def _gat_edges(x, src, dst, W, a_s, a_d, b, heads, ch):
    h = _matmul(x, W).reshape(N, heads, ch)
    es = jnp.sum(h * a_s[None], axis=-1)
    ed = jnp.sum(h * a_d[None], axis=-1)
    e = jax.nn.leaky_relu(es[src] + ed[dst], 0.2)
    emax = jax.ops.segment_max(e, dst, num_segments=N)
    emax = jnp.where(jnp.isfinite(emax), emax, 0.0)
    ex = jnp.exp(e - emax[dst])
    den = jax.ops.segment_sum(ex, dst, num_segments=N)
    alpha = ex / (den[dst] + 1e-16)
    out = jax.ops.segment_sum(h[src] * alpha[:, :, None], dst, num_segments=N)
    return out.reshape(N, heads * ch) + b


def kernel(d_sim, m_sim, diseases, mirnas, W_d, W_m, W_lin1, b_lin1, W1, a_s1, a_d1, b1, W2, a_s2, a_d2, b2, Wp, bp, ln_g, ln_b, Wl, bl, Wd1, bd1, Wm1, bm1, Wpred, bpred):
    z_d = _matmul(d_sim, W_d)
    z_m = _matmul(m_sim, W_m)
    feats = jnp.concatenate([z_d, z_m], axis=0)
    src = diseases
    dst = mirnas

    x1 = jax.nn.relu(_matmul(feats, W_lin1) + b_lin1)
    xg = jax.nn.relu(_gat_edges(feats, src, dst, W1, a_s1, a_d1, b1, 8, 256))
    x2 = _gat_edges(xg, src, dst, W2, a_s2, a_d2, b2, 1, 256)
    final = jnp.concatenate([x1, x2], axis=1)

    Wp_pad = jnp.pad(Wp, ((0, 0), (0, 124)))
    bp_pad = jnp.pad(bp, (0, 124))
    g = _matmul(final, Wp_pad) + bp_pad
    key, v50, tie = _knn_select(g)
    outf = _knn_mix(key, v50, tie, final, ln_g, ln_b, Wl, bl)
    h_d = jnp.concatenate([outf[:ND], feats[:ND]], axis=1)
    h_m = jnp.concatenate([outf[ND:], feats[ND:]], axis=1)
    h_d = jax.nn.elu(_matmul(h_d, Wd1) + bd1)
    h_m = jax.nn.elu(_matmul(h_m, Wm1) + bm1)
    h = jnp.concatenate([h_d, h_m], axis=0)
    hc = jnp.concatenate([h[diseases], h[mirnas]], axis=1)
    Wpred_pad = jnp.pad(Wpred, ((0, 0), (0, 127)))
    logits = _matmul(hc, Wpred_pad)[:, :1]
    return jax.nn.sigmoid(logits + bpred)


# speedup vs baseline: 8.2289x; 3.6898x over previous
import functools

import jax
import jax.numpy as jnp
from jax import lax
from jax.experimental import pallas as pl
from jax.experimental.pallas import tpu as pltpu
from jax.experimental.pallas import tpu_sc as plsc

_NW = 32
_EPW = 1024

ND = 1024
NM = 3072
N = ND + NM
E = 32768
HID = 64

_HI = jax.lax.Precision.HIGHEST


def _mm_body(a_ref, b_ref, o_ref):
    o_ref[...] = jnp.dot(a_ref[...], b_ref[...],
                         preferred_element_type=jnp.float32, precision=_HI)


def _matmul(a, b, bm=256):
    M, K = a.shape
    _, Nn = b.shape
    return pl.pallas_call(
        _mm_body,
        grid=(M // bm,),
        in_specs=[pl.BlockSpec((bm, K), lambda i: (i, 0)),
                  pl.BlockSpec((K, Nn), lambda i: (0, 0))],
        out_specs=pl.BlockSpec((bm, Nn), lambda i: (i, 0)),
        out_shape=jax.ShapeDtypeStruct((M, Nn), jnp.float32),
    )(a, b)


_K = 50


def _f32_sortkey(x):
    b = jax.lax.bitcast_convert_type(x, jnp.int32)
    imin = jnp.full(b.shape, -(2 ** 31), jnp.int32)
    return jnp.where(b >= 0, b, jnp.bitwise_xor(jnp.bitwise_not(b), imin))


def _knn_sel_body(g_ref, gt_ref, key_ref, v50_ref, tie_ref):
    gblk = g_ref[...]
    gt = gt_ref[...]
    dot = jnp.dot(gblk, gt, preferred_element_type=jnp.float32, precision=_HI)
    sq_row = jnp.sum(gblk * gblk, axis=1, keepdims=True)
    sq_col = jnp.sum(gt * gt, axis=0, keepdims=True)
    d2 = sq_row + sq_col - 2.0 * dot
    key = _f32_sortkey(d2)
    key_ref[...] = key
    lo = jnp.min(key, axis=1, keepdims=True)
    hi = jnp.max(key, axis=1, keepdims=True)

    def vstep(_, carry):
        lo, hi = carry
        mid = (lo >> 1) + (hi >> 1) + (lo & hi & 1)
        cnt = jnp.sum((key <= mid).astype(jnp.int32), axis=1, keepdims=True)
        sel = cnt >= _K
        return jnp.where(sel, lo, mid + 1), jnp.where(sel, mid, hi)

    lo, hi = jax.lax.fori_loop(0, 32, vstep, (lo, hi))
    v50 = lo
    v50_ref[...] = v50
    r = _K - jnp.sum((key < v50).astype(jnp.int32), axis=1, keepdims=True)
    eq = (key == v50)
    col = jax.lax.broadcasted_iota(jnp.int32, key.shape, 1)
    tlo = jnp.zeros_like(v50)
    thi = jnp.full_like(v50, key.shape[1] - 1)

    def tstep(_, carry):
        tlo, thi = carry
        tmid = (tlo + thi) >> 1
        cnt = jnp.sum((eq & (col <= tmid)).astype(jnp.int32), axis=1,
                      keepdims=True)
        sel = cnt >= r
        return jnp.where(sel, tlo, tmid + 1), jnp.where(sel, tmid, thi)

    tlo, thi = jax.lax.fori_loop(0, 12, tstep, (tlo, thi))
    tie_ref[...] = tlo


def _knn_select(g, bm=256):
    gt = g.T
    return pl.pallas_call(
        _knn_sel_body,
        grid=(N // bm,),
        in_specs=[pl.BlockSpec((bm, g.shape[1]), lambda i: (i, 0)),
                  pl.BlockSpec((g.shape[1], N), lambda i: (0, 0))],
        out_specs=[pl.BlockSpec((bm, N), lambda i: (i, 0)),
                   pl.BlockSpec((bm, 1), lambda i: (i, 0)),
                   pl.BlockSpec((bm, 1), lambda i: (i, 0))],
        out_shape=[jax.ShapeDtypeStruct((N, N), jnp.int32),
                   jax.ShapeDtypeStruct((N, 1), jnp.int32),
                   jax.ShapeDtypeStruct((N, 1), jnp.int32)],
    )(g, gt)


def _knn_mix_body(key_ref, v50_ref, tie_ref, fin_ref, finblk_ref, lng_ref,
                  lnb_ref, wla_ref, wlb_ref, bl_ref, o_ref):
    key = key_ref[...]
    v50 = v50_ref[...]
    col = jax.lax.broadcasted_iota(jnp.int32, key.shape, 1)
    mask = ((key < v50) | ((key == v50) & (col <= tie_ref[...]))).astype(
        jnp.float32)
    sim = jnp.dot(mask, fin_ref[...], preferred_element_type=jnp.float32,
                  precision=_HI) * (1.0 / _K)
    mu = jnp.mean(sim, axis=1, keepdims=True)
    var = jnp.mean((sim - mu) ** 2, axis=1, keepdims=True)
    sim_n = (sim - mu) / jnp.sqrt(var + 1e-5) * lng_ref[...] + lnb_ref[...]
    out = jnp.dot(finblk_ref[...], wla_ref[...],
                  preferred_element_type=jnp.float32, precision=_HI)
    out += jnp.dot(sim_n, wlb_ref[...], preferred_element_type=jnp.float32,
                   precision=_HI)
    o_ref[...] = out + bl_ref[...]


def _knn_mix(key, v50, tie, final, ln_g, ln_b, Wl, bl, bm=256):
    F = final.shape[1]
    out_dim = Wl.shape[1]
    return pl.pallas_call(
        _knn_mix_body,
        grid=(N // bm,),
        in_specs=[pl.BlockSpec((bm, N), lambda i: (i, 0)),
                  pl.BlockSpec((bm, 1), lambda i: (i, 0)),
                  pl.BlockSpec((bm, 1), lambda i: (i, 0)),
                  pl.BlockSpec((N, F), lambda i: (0, 0)),
                  pl.BlockSpec((bm, F), lambda i: (i, 0)),
                  pl.BlockSpec((1, F), lambda i: (0, 0)),
                  pl.BlockSpec((1, F), lambda i: (0, 0)),
                  pl.BlockSpec((F, out_dim), lambda i: (0, 0)),
                  pl.BlockSpec((F, out_dim), lambda i: (0, 0)),
                  pl.BlockSpec((1, out_dim), lambda i: (0, 0))],
        out_specs=pl.BlockSpec((bm, out_dim), lambda i: (i, 0)),
        out_shape=jax.ShapeDtypeStruct((N, out_dim), jnp.float32),
    )(key, v50, tie, final, final, ln_g.reshape(1, F), ln_b.reshape(1, F),
      Wl[:F], Wl[F:], bl.reshape(1, out_dim))


def _mm_chunks_body(a_ref, b_ref, o_ref):
    o_ref[0] = jnp.dot(a_ref[...], b_ref[...],
                       preferred_element_type=jnp.float32, precision=_HI)


def _matmul_chunks(a, b, chunks, cw=256, bm=512):
    M, K = a.shape
    return pl.pallas_call(
        _mm_chunks_body,
        grid=(chunks, M // bm),
        in_specs=[pl.BlockSpec((bm, K), lambda c, i: (i, 0)),
                  pl.BlockSpec((K, cw), lambda c, i: (0, c))],
        out_specs=pl.BlockSpec((1, bm, cw), lambda c, i: (c, i, 0)),
        out_shape=jax.ShapeDtypeStruct((chunks, M, cw), jnp.float32),
    )(a, b)


def _sc_mesh():
    return plsc.VectorSubcoreMesh(core_axis_name="c", subcore_axis_name="s")


def _sc_gat_scores(ES, ED, src_r, dst_r, zrow):
    G = 128

    def body(es_h, ed_h, src_h, dst_h, z_h, ex_h, den_h,
             acc_sh, src_v, dst_v, a_v, b_v, ex_v, sem):
        cid = lax.axis_index("c")
        sid = lax.axis_index("s")
        wid = sid * 2 + cid
        base = wid * _EPW
        pltpu.sync_copy(src_h.at[wid], src_v)
        pltpu.sync_copy(dst_h.at[wid], dst_v)
        pltpu.sync_copy(z_h, acc_sh.at[pl.ds(sid * 256, 256)])
        plsc.subcore_barrier()

        def group(g, c):
            pltpu.async_copy(es_h.at[src_v.at[g]], a_v, sem).wait()
            pltpu.async_copy(ed_h.at[dst_v.at[g]], b_v, sem).wait()

            def lane(i, c2):
                for cc in range(8):
                    v = (a_v[i, pl.ds(cc * 16, 16)]
                         + b_v[i, pl.ds(cc * 16, 16)])
                    v = jnp.where(v > 0, v, v * 0.2)
                    ex_v[i, pl.ds(cc * 16, 16)] = jnp.exp(v)
                return c2
            lax.fori_loop(0, G, lane, 0)
            pltpu.sync_copy(ex_v, ex_h.at[pl.ds(base + g * G, G)])
            pltpu.sync_copy(ex_v, acc_sh.at[dst_v.at[g]], add=True)
            return c
        lax.fori_loop(0, _EPW // G, group, 0)
        plsc.subcore_barrier()
        pltpu.sync_copy(acc_sh.at[pl.ds(sid * 256, 256)],
                        den_h.at[pl.ds(cid * N + sid * 256, 256)])

    fn = pl.kernel(
        body,
        out_type=[jax.ShapeDtypeStruct((E, 128), jnp.float32),
                  jax.ShapeDtypeStruct((2 * N, 128), jnp.float32)],
        mesh=_sc_mesh(),
        compiler_params=pltpu.CompilerParams(use_tc_tiling_on_sc=False),
        scratch_types=[pltpu.VMEM_SHARED((N, 128), jnp.float32),
                       pltpu.VMEM((8, G), jnp.int32),
                       pltpu.VMEM((8, G), jnp.int32),
                       pltpu.VMEM((G, 128), jnp.float32),
                       pltpu.VMEM((G, 128), jnp.float32),
                       pltpu.VMEM((G, 128), jnp.float32),
                       pltpu.SemaphoreType.DMA],
    )
    return fn(ES, ED, src_r, dst_r, zrow)


def _deninv_body(p0_ref, p1_ref, o_ref):
    o_ref[...] = 1.0 / (p0_ref[...] + p1_ref[...] + 1e-16)


def _den_inv(den_part, bm=512):
    return pl.pallas_call(
        _deninv_body,
        grid=(N // bm,),
        in_specs=[pl.BlockSpec((bm, 128), lambda i: (i, 0)),
                  pl.BlockSpec((bm, 128), lambda i: (i + N // bm, 0))],
        out_specs=pl.BlockSpec((bm, 128), lambda i: (i, 0)),
        out_shape=jax.ShapeDtypeStruct((N, 128), jnp.float32),
    )(den_part, den_part)


def _sc_gat_aggregate(h_chunk, exb, src_r, dst_r, zrow):
    G = 128

    def body(h_h, exb_h, src_h, dst_h, z_h, out_h,
             acc_sh, src_v, dst_v, exb_v, rows_v, sem):
        cid = lax.axis_index("c")
        sid = lax.axis_index("s")
        wid = sid * 2 + cid
        base = wid * _EPW
        pltpu.sync_copy(src_h.at[wid], src_v)
        pltpu.sync_copy(dst_h.at[wid], dst_v)
        pltpu.sync_copy(exb_h.at[pl.ds(base, _EPW)], exb_v)
        pltpu.sync_copy(z_h, acc_sh.at[pl.ds(sid * 256, 256)])
        plsc.subcore_barrier()

        def group(g, c):
            pltpu.async_copy(h_h.at[src_v.at[g]], rows_v, sem).wait()

            def scale(j, c2):
                ev = exb_v[g * G + j]
                for cc in range(16):
                    rows_v[j, pl.ds(cc * 16, 16)] = (
                        rows_v[j, pl.ds(cc * 16, 16)] * ev)
                return c2
            lax.fori_loop(0, G, scale, 0)
            pltpu.sync_copy(rows_v, acc_sh.at[dst_v.at[g]], add=True)
            return c
        lax.fori_loop(0, _EPW // G, group, 0)
        plsc.subcore_barrier()
        pltpu.sync_copy(acc_sh.at[pl.ds(sid * 256, 256)],
                        out_h.at[pl.ds(cid * N + sid * 256, 256)])

    fn = pl.kernel(
        body,
        out_type=jax.ShapeDtypeStruct((2 * N, 256), jnp.float32),
        mesh=_sc_mesh(),
        compiler_params=pltpu.CompilerParams(use_tc_tiling_on_sc=False),
        scratch_types=[pltpu.VMEM_SHARED((N, 256), jnp.float32),
                       pltpu.VMEM((8, G), jnp.int32),
                       pltpu.VMEM((8, G), jnp.int32),
                       pltpu.VMEM((_EPW, 16), jnp.float32),
                       pltpu.VMEM((G, 256), jnp.float32),
                       pltpu.SemaphoreType.DMA],
    )
    return fn(h_chunk, exb, src_r, dst_r, zrow)


def _gat_sc(h_chunks, es2, ed2, src, dst):
    chunks = h_chunks.shape[0]
    src_r = src.reshape(_NW, 8, 128)
    dst_r = dst.reshape(_NW, 8, 128)
    z16 = jnp.zeros((256, 128), jnp.float32)
    z256 = jnp.zeros((256, 256), jnp.float32)
    ex, den_part = _sc_gat_scores(es2, ed2, src_r, dst_r, z16)
    inv = _den_inv(den_part)
    p0s, p1s = [], []
    for cc in range(chunks):
        exb = jnp.broadcast_to(ex[:, cc:cc + 1], (E, 16))
        p = _sc_gat_aggregate(h_chunks[cc], exb, src_r, dst_r, z256)
        p0s.append(p[:N])
        p1s.append(p[N:])
    return jnp.concatenate(p0s, 1), jnp.concatenate(p1s, 1), inv


def _relu_add_mm_body(p0_ref, p1_ref, inv_ref, b_ref, w_ref, o_ref):
    xg = jax.nn.relu((p0_ref[...] + p1_ref[...]) * inv_ref[...] + b_ref[...])
    o_ref[...] = jnp.dot(xg, w_ref[...], preferred_element_type=jnp.float32,
                         precision=_HI)


def _relu_add_mm(p0, p1, invb, b, w, bm=512):
    M, K = p0.shape
    Nn = w.shape[1]
    return pl.pallas_call(
        _relu_add_mm_body,
        grid=(M // bm,),
        in_specs=[pl.BlockSpec((bm, K), lambda i: (i, 0)),
                  pl.BlockSpec((bm, K), lambda i: (i, 0)),
                  pl.BlockSpec((bm, K), lambda i: (i, 0)),
                  pl.BlockSpec((1, K), lambda i: (0, 0)),
                  pl.BlockSpec((K, Nn), lambda i: (0, 0))],
        out_specs=pl.BlockSpec((bm, Nn), lambda i: (i, 0)),
        out_shape=jax.ShapeDtypeStruct((M, Nn), jnp.float32),
    )(p0, p1, invb, b.reshape(1, K), w)


def _final_body(x1_ref, bl1_ref, p0_ref, p1_ref, inv_ref, b2_ref, o_ref):
    o_ref[:, :256] = jax.nn.relu(x1_ref[...] + bl1_ref[...])
    o_ref[:, 256:] = (p0_ref[...] + p1_ref[...]) * inv_ref[...] + b2_ref[...]


def _final_assemble(x1raw, b_lin1, p0, p1, invb2, b2, bm=512):
    return pl.pallas_call(
        _final_body,
        grid=(N // bm,),
        in_specs=[pl.BlockSpec((bm, 256), lambda i: (i, 0)),
                  pl.BlockSpec((1, 256), lambda i: (0, 0)),
                  pl.BlockSpec((bm, 256), lambda i: (i, 0)),
                  pl.BlockSpec((bm, 256), lambda i: (i, 0)),
                  pl.BlockSpec((bm, 256), lambda i: (i, 0)),
                  pl.BlockSpec((1, 256), lambda i: (0, 0))],
        out_specs=pl.BlockSpec((bm, 512), lambda i: (i, 0)),
        out_shape=jax.ShapeDtypeStruct((N, 512), jnp.float32),
    )(x1raw, b_lin1.reshape(1, 256), p0, p1, invb2, b2.reshape(1, 256))


def kernel(d_sim, m_sim, diseases, mirnas, W_d, W_m, W_lin1, b_lin1, W1, a_s1, a_d1, b1, W2, a_s2, a_d2, b2, Wp, bp, ln_g, ln_b, Wl, bl, Wd1, bd1, Wm1, bm1, Wpred, bpred):
    z_d = _matmul(d_sim, W_d)
    z_m = _matmul(m_sim, W_m)
    feats = jnp.concatenate([z_d, z_m], axis=0)
    src = diseases.astype(jnp.int32)
    dst = mirnas.astype(jnp.int32)

    x1raw = _matmul(feats, W_lin1)

    eye8 = jnp.eye(8, dtype=jnp.float32)
    A_s1 = (eye8[:, None, :] * a_s1[:, :, None]).reshape(2048, 8)
    A_d1 = (eye8[:, None, :] * a_d1[:, :, None]).reshape(2048, 8)
    A_sd1 = jnp.pad(jnp.concatenate([A_s1, A_d1], axis=1), ((0, 0), (0, 112)))
    W1A = _matmul(W1, A_sd1, bm=64)
    esd1 = _matmul(feats, W1A)
    ES1 = jnp.pad(jnp.concatenate([esd1[:, :8], esd1[:, :8]], axis=1),
                  ((0, 0), (0, 112)))
    ED1 = jnp.pad(jnp.concatenate([esd1[:, 8:16], esd1[:, 8:16]], axis=1),
                  ((0, 0), (0, 112)))
    h1c = _matmul_chunks(feats, W1, 8)
    p0, p1g, inv1 = _gat_sc(h1c, ES1, ED1, src, dst)
    invb1 = jnp.repeat(inv1[:, :8], 256, axis=1)

    h2 = _relu_add_mm(p0, p1g, invb1, b1, W2)
    A2 = jnp.pad(jnp.concatenate([a_s2.T, a_d2.T], axis=1),
                 ((0, 0), (0, 126)))
    esd2 = _matmul(h2, A2)
    ES2 = jnp.pad(jnp.broadcast_to(esd2[:, 0:1], (N, 16)),
                  ((0, 0), (0, 112)))
    ED2 = jnp.pad(jnp.broadcast_to(esd2[:, 1:2], (N, 16)),
                  ((0, 0), (0, 112)))
    q0, q1, inv2 = _gat_sc(h2.reshape(1, N, 256), ES2, ED2, src, dst)
    invb2 = jnp.broadcast_to(inv2[:, 0:1], (N, 256))

    final = _final_assemble(x1raw, b_lin1, q0, q1, invb2, b2)

    Wp_pad = jnp.pad(Wp, ((0, 0), (0, 124)))
    bp_pad = jnp.pad(bp, (0, 124))
    g = _matmul(final, Wp_pad) + bp_pad
    key, v50, tie = _knn_select(g)
    outf = _knn_mix(key, v50, tie, final, ln_g, ln_b, Wl, bl)
    h_d = jnp.concatenate([outf[:ND], feats[:ND]], axis=1)
    h_m = jnp.concatenate([outf[ND:], feats[ND:]], axis=1)
    h_d = jax.nn.elu(_matmul(h_d, Wd1) + bd1)
    h_m = jax.nn.elu(_matmul(h_m, Wm1) + bm1)
    h = jnp.concatenate([h_d, h_m], axis=0)
    hc = jnp.concatenate([h[diseases], h[mirnas]], axis=1)
    Wpred_pad = jnp.pad(Wpred, ((0, 0), (0, 127)))
    logits = _matmul(hc, Wpred_pad)[:, :1]
    return jax.nn.sigmoid(logits + bpred)


# speedup vs baseline: 8.8269x; 1.0727x over previous
import functools

import jax
import jax.numpy as jnp
from jax import lax
from jax.experimental import pallas as pl
from jax.experimental.pallas import tpu as pltpu
from jax.experimental.pallas import tpu_sc as plsc

_NW = 32
_EPW = 1024

ND = 1024
NM = 3072
N = ND + NM
E = 32768
HID = 64

_HI = jax.lax.Precision.HIGHEST


def _mm_body(a_ref, b_ref, o_ref):
    o_ref[...] = jnp.dot(a_ref[...], b_ref[...],
                         preferred_element_type=jnp.float32, precision=_HI)


def _matmul(a, b, bm=256):
    M, K = a.shape
    _, Nn = b.shape
    return pl.pallas_call(
        _mm_body,
        grid=(M // bm,),
        in_specs=[pl.BlockSpec((bm, K), lambda i: (i, 0)),
                  pl.BlockSpec((K, Nn), lambda i: (0, 0))],
        out_specs=pl.BlockSpec((bm, Nn), lambda i: (i, 0)),
        out_shape=jax.ShapeDtypeStruct((M, Nn), jnp.float32),
    )(a, b)


_K = 50


def _f32_sortkey(x):
    b = jax.lax.bitcast_convert_type(x, jnp.int32)
    imin = jnp.full(b.shape, -(2 ** 31), jnp.int32)
    return jnp.where(b >= 0, b, jnp.bitwise_xor(jnp.bitwise_not(b), imin))


def _knn_sel_body(g_ref, gt_ref, key_ref, v50_ref, tie_ref):
    gblk = g_ref[...]
    gt = gt_ref[...]
    dot = jnp.dot(gblk, gt, preferred_element_type=jnp.float32, precision=_HI)
    sq_row = jnp.sum(gblk * gblk, axis=1, keepdims=True)
    sq_col = jnp.sum(gt * gt, axis=0, keepdims=True)
    d2 = sq_row + sq_col - 2.0 * dot
    key = _f32_sortkey(d2)
    key_ref[...] = key
    lo = jnp.min(key, axis=1, keepdims=True)
    hi = jnp.max(key, axis=1, keepdims=True)

    def vstep(_, carry):
        lo, hi = carry
        mid = (lo >> 1) + (hi >> 1) + (lo & hi & 1)
        cnt = jnp.sum((key <= mid).astype(jnp.int32), axis=1, keepdims=True)
        sel = cnt >= _K
        return jnp.where(sel, lo, mid + 1), jnp.where(sel, mid, hi)

    lo, hi = jax.lax.fori_loop(0, 32, vstep, (lo, hi))
    v50 = lo
    v50_ref[...] = v50
    r = _K - jnp.sum((key < v50).astype(jnp.int32), axis=1, keepdims=True)
    eq = (key == v50)
    col = jax.lax.broadcasted_iota(jnp.int32, key.shape, 1)
    tlo = jnp.zeros_like(v50)
    thi = jnp.full_like(v50, key.shape[1] - 1)

    def tstep(_, carry):
        tlo, thi = carry
        tmid = (tlo + thi) >> 1
        cnt = jnp.sum((eq & (col <= tmid)).astype(jnp.int32), axis=1,
                      keepdims=True)
        sel = cnt >= r
        return jnp.where(sel, tlo, tmid + 1), jnp.where(sel, tmid, thi)

    tlo, thi = jax.lax.fori_loop(0, 12, tstep, (tlo, thi))
    tie_ref[...] = tlo


def _knn_select(g, bm=256):
    gt = g.T
    return pl.pallas_call(
        _knn_sel_body,
        grid=(N // bm,),
        in_specs=[pl.BlockSpec((bm, g.shape[1]), lambda i: (i, 0)),
                  pl.BlockSpec((g.shape[1], N), lambda i: (0, 0))],
        out_specs=[pl.BlockSpec((bm, N), lambda i: (i, 0)),
                   pl.BlockSpec((bm, 1), lambda i: (i, 0)),
                   pl.BlockSpec((bm, 1), lambda i: (i, 0))],
        out_shape=[jax.ShapeDtypeStruct((N, N), jnp.int32),
                   jax.ShapeDtypeStruct((N, 1), jnp.int32),
                   jax.ShapeDtypeStruct((N, 1), jnp.int32)],
    )(g, gt)


def _knn_mix_body(key_ref, v50_ref, tie_ref, fin_ref, finblk_ref, lng_ref,
                  lnb_ref, wla_ref, wlb_ref, bl_ref, o_ref):
    key = key_ref[...]
    v50 = v50_ref[...]
    col = jax.lax.broadcasted_iota(jnp.int32, key.shape, 1)
    mask = ((key < v50) | ((key == v50) & (col <= tie_ref[...]))).astype(
        jnp.float32)
    sim = jnp.dot(mask, fin_ref[...], preferred_element_type=jnp.float32,
                  precision=_HI) * (1.0 / _K)
    mu = jnp.mean(sim, axis=1, keepdims=True)
    var = jnp.mean((sim - mu) ** 2, axis=1, keepdims=True)
    sim_n = (sim - mu) / jnp.sqrt(var + 1e-5) * lng_ref[...] + lnb_ref[...]
    out = jnp.dot(finblk_ref[...], wla_ref[...],
                  preferred_element_type=jnp.float32, precision=_HI)
    out += jnp.dot(sim_n, wlb_ref[...], preferred_element_type=jnp.float32,
                   precision=_HI)
    o_ref[...] = out + bl_ref[...]


def _knn_mix(key, v50, tie, final, ln_g, ln_b, Wl, bl, bm=256):
    F = final.shape[1]
    out_dim = Wl.shape[1]
    return pl.pallas_call(
        _knn_mix_body,
        grid=(N // bm,),
        in_specs=[pl.BlockSpec((bm, N), lambda i: (i, 0)),
                  pl.BlockSpec((bm, 1), lambda i: (i, 0)),
                  pl.BlockSpec((bm, 1), lambda i: (i, 0)),
                  pl.BlockSpec((N, F), lambda i: (0, 0)),
                  pl.BlockSpec((bm, F), lambda i: (i, 0)),
                  pl.BlockSpec((1, F), lambda i: (0, 0)),
                  pl.BlockSpec((1, F), lambda i: (0, 0)),
                  pl.BlockSpec((F, out_dim), lambda i: (0, 0)),
                  pl.BlockSpec((F, out_dim), lambda i: (0, 0)),
                  pl.BlockSpec((1, out_dim), lambda i: (0, 0))],
        out_specs=pl.BlockSpec((bm, out_dim), lambda i: (i, 0)),
        out_shape=jax.ShapeDtypeStruct((N, out_dim), jnp.float32),
    )(key, v50, tie, final, final, ln_g.reshape(1, F), ln_b.reshape(1, F),
      Wl[:F], Wl[F:], bl.reshape(1, out_dim))


def _mm_chunks_body(a_ref, b_ref, o_ref):
    o_ref[0] = jnp.dot(a_ref[...], b_ref[...],
                       preferred_element_type=jnp.float32, precision=_HI)


def _matmul_chunks(a, b, chunks, cw=256, bm=512):
    M, K = a.shape
    return pl.pallas_call(
        _mm_chunks_body,
        grid=(chunks, M // bm),
        in_specs=[pl.BlockSpec((bm, K), lambda c, i: (i, 0)),
                  pl.BlockSpec((K, cw), lambda c, i: (0, c))],
        out_specs=pl.BlockSpec((1, bm, cw), lambda c, i: (c, i, 0)),
        out_shape=jax.ShapeDtypeStruct((chunks, M, cw), jnp.float32),
    )(a, b)


def _sc_mesh():
    return plsc.VectorSubcoreMesh(core_axis_name="c", subcore_axis_name="s")


def _sc_gat_scores(ES, ED, src_r, dst_r, zrow):
    G = 128

    def body(es_h, ed_h, src_h, dst_h, z_h, ex_h, den_h,
             acc_sh, src_v, dst_v, a_v, b_v, ex_v, sem):
        cid = lax.axis_index("c")
        sid = lax.axis_index("s")
        wid = sid * 2 + cid
        base = wid * _EPW
        pltpu.sync_copy(src_h.at[wid], src_v)
        pltpu.sync_copy(dst_h.at[wid], dst_v)
        pltpu.sync_copy(z_h, acc_sh.at[pl.ds(sid * 256, 256)])
        plsc.subcore_barrier()

        def group(g, c):
            pltpu.async_copy(es_h.at[src_v.at[g]], a_v, sem).wait()
            pltpu.async_copy(ed_h.at[dst_v.at[g]], b_v, sem).wait()

            def lane(i, c2):
                for cc in range(8):
                    v = (a_v[i, pl.ds(cc * 16, 16)]
                         + b_v[i, pl.ds(cc * 16, 16)])
                    v = jnp.where(v > 0, v, v * 0.2)
                    ex_v[i, pl.ds(cc * 16, 16)] = jnp.exp(v)
                return c2
            lax.fori_loop(0, G, lane, 0)
            pltpu.sync_copy(ex_v, ex_h.at[pl.ds(base + g * G, G)])
            pltpu.sync_copy(ex_v, acc_sh.at[dst_v.at[g]], add=True)
            return c
        lax.fori_loop(0, _EPW // G, group, 0)
        plsc.subcore_barrier()
        pltpu.sync_copy(acc_sh.at[pl.ds(sid * 256, 256)],
                        den_h.at[pl.ds(cid * N + sid * 256, 256)])

    fn = pl.kernel(
        body,
        out_type=[jax.ShapeDtypeStruct((E, 128), jnp.float32),
                  jax.ShapeDtypeStruct((2 * N, 128), jnp.float32)],
        mesh=_sc_mesh(),
        compiler_params=pltpu.CompilerParams(use_tc_tiling_on_sc=False),
        scratch_types=[pltpu.VMEM_SHARED((N, 128), jnp.float32),
                       pltpu.VMEM((8, G), jnp.int32),
                       pltpu.VMEM((8, G), jnp.int32),
                       pltpu.VMEM((G, 128), jnp.float32),
                       pltpu.VMEM((G, 128), jnp.float32),
                       pltpu.VMEM((G, 128), jnp.float32),
                       pltpu.SemaphoreType.DMA],
    )
    return fn(ES, ED, src_r, dst_r, zrow)


def _deninv_body(p0_ref, p1_ref, o_ref):
    o_ref[...] = 1.0 / (p0_ref[...] + p1_ref[...] + 1e-16)


def _den_inv(den_part, bm=512):
    return pl.pallas_call(
        _deninv_body,
        grid=(N // bm,),
        in_specs=[pl.BlockSpec((bm, 128), lambda i: (i, 0)),
                  pl.BlockSpec((bm, 128), lambda i: (i + N // bm, 0))],
        out_specs=pl.BlockSpec((bm, 128), lambda i: (i, 0)),
        out_shape=jax.ShapeDtypeStruct((N, 128), jnp.float32),
    )(den_part, den_part)


def _sc_gat_aggregate(h_chunk, exb, src_r, dst_r, zrow):
    G = 128

    def body(h_h, exb_h, src_h, dst_h, z_h, out_h,
             acc_sh, src_v, dst_v, exb_v, rows_v, sem):
        cid = lax.axis_index("c")
        sid = lax.axis_index("s")
        wid = sid * 2 + cid
        base = wid * _EPW
        pltpu.sync_copy(src_h.at[wid], src_v)
        pltpu.sync_copy(dst_h.at[wid], dst_v)
        pltpu.sync_copy(exb_h.at[pl.ds(base, _EPW)], exb_v)
        pltpu.sync_copy(z_h, acc_sh.at[pl.ds(sid * 256, 256)])
        plsc.subcore_barrier()

        def group(g, c):
            pltpu.async_copy(h_h.at[src_v.at[g]], rows_v, sem).wait()

            def scale(j, c2):
                ev = exb_v[g * G + j]
                for cc in range(16):
                    rows_v[j, pl.ds(cc * 16, 16)] = (
                        rows_v[j, pl.ds(cc * 16, 16)] * ev)
                return c2
            lax.fori_loop(0, G, scale, 0)
            pltpu.sync_copy(rows_v, acc_sh.at[dst_v.at[g]], add=True)
            return c
        lax.fori_loop(0, _EPW // G, group, 0)
        plsc.subcore_barrier()
        pltpu.sync_copy(acc_sh.at[pl.ds(sid * 256, 256)],
                        out_h.at[pl.ds(cid * N + sid * 256, 256)])

    fn = pl.kernel(
        body,
        out_type=jax.ShapeDtypeStruct((2 * N, 256), jnp.float32),
        mesh=_sc_mesh(),
        compiler_params=pltpu.CompilerParams(use_tc_tiling_on_sc=False),
        scratch_types=[pltpu.VMEM_SHARED((N, 256), jnp.float32),
                       pltpu.VMEM((8, G), jnp.int32),
                       pltpu.VMEM((8, G), jnp.int32),
                       pltpu.VMEM((_EPW, 16), jnp.float32),
                       pltpu.VMEM((G, 256), jnp.float32),
                       pltpu.SemaphoreType.DMA],
    )
    return fn(h_chunk, exb, src_r, dst_r, zrow)


def _gat_sc(h_chunks, es2, ed2, src, dst):
    chunks = h_chunks.shape[0]
    src_r = src.reshape(_NW, 8, 128)
    dst_r = dst.reshape(_NW, 8, 128)
    z16 = jnp.zeros((256, 128), jnp.float32)
    z256 = jnp.zeros((256, 256), jnp.float32)
    ex, den_part = _sc_gat_scores(es2, ed2, src_r, dst_r, z16)
    inv = _den_inv(den_part)
    p0s, p1s = [], []
    for cc in range(chunks):
        exb = jnp.broadcast_to(ex[:, cc:cc + 1], (E, 16))
        p = _sc_gat_aggregate(h_chunks[cc], exb, src_r, dst_r, z256)
        p0s.append(p[:N])
        p1s.append(p[N:])
    return jnp.concatenate(p0s, 1), jnp.concatenate(p1s, 1), inv


def _sc_edge_gather(h_pad, src_r, dst_r):
    G = 128

    def body(h_h, src_h, dst_h, o1_h, o2_h, src_v, dst_v, a_v, b_v, sem):
        cid = lax.axis_index("c")
        sid = lax.axis_index("s")
        wid = sid * 2 + cid
        base = wid * _EPW
        pltpu.sync_copy(src_h.at[wid], src_v)
        pltpu.sync_copy(dst_h.at[wid], dst_v)

        def group(g, c):
            pltpu.async_copy(h_h.at[src_v.at[g]], a_v, sem).wait()
            pltpu.async_copy(h_h.at[dst_v.at[g]], b_v, sem).wait()
            pltpu.sync_copy(a_v, o1_h.at[pl.ds(base + g * G, G)])
            pltpu.sync_copy(b_v, o2_h.at[pl.ds(base + g * G, G)])
            return c
        lax.fori_loop(0, _EPW // G, group, 0)

    fn = pl.kernel(
        body,
        out_type=[jax.ShapeDtypeStruct((E, 128), jnp.float32),
                  jax.ShapeDtypeStruct((E, 128), jnp.float32)],
        mesh=_sc_mesh(),
        compiler_params=pltpu.CompilerParams(use_tc_tiling_on_sc=False),
        scratch_types=[pltpu.VMEM((8, G), jnp.int32),
                       pltpu.VMEM((8, G), jnp.int32),
                       pltpu.VMEM((G, 128), jnp.float32),
                       pltpu.VMEM((G, 128), jnp.float32),
                       pltpu.SemaphoreType.DMA],
    )
    return fn(h_pad, src_r, dst_r)


def _relu_add_mm_body(p0_ref, p1_ref, inv_ref, b_ref, w_ref, o_ref):
    xg = jax.nn.relu((p0_ref[...] + p1_ref[...]) * inv_ref[...] + b_ref[...])
    o_ref[...] = jnp.dot(xg, w_ref[...], preferred_element_type=jnp.float32,
                         precision=_HI)


def _relu_add_mm(p0, p1, invb, b, w, bm=512):
    M, K = p0.shape
    Nn = w.shape[1]
    return pl.pallas_call(
        _relu_add_mm_body,
        grid=(M // bm,),
        in_specs=[pl.BlockSpec((bm, K), lambda i: (i, 0)),
                  pl.BlockSpec((bm, K), lambda i: (i, 0)),
                  pl.BlockSpec((bm, K), lambda i: (i, 0)),
                  pl.BlockSpec((1, K), lambda i: (0, 0)),
                  pl.BlockSpec((K, Nn), lambda i: (0, 0))],
        out_specs=pl.BlockSpec((bm, Nn), lambda i: (i, 0)),
        out_shape=jax.ShapeDtypeStruct((M, Nn), jnp.float32),
    )(p0, p1, invb, b.reshape(1, K), w)


def _final_body(x1_ref, bl1_ref, p0_ref, p1_ref, inv_ref, b2_ref, o_ref):
    o_ref[:, :256] = jax.nn.relu(x1_ref[...] + bl1_ref[...])
    o_ref[:, 256:] = (p0_ref[...] + p1_ref[...]) * inv_ref[...] + b2_ref[...]


def _final_assemble(x1raw, b_lin1, p0, p1, invb2, b2, bm=512):
    return pl.pallas_call(
        _final_body,
        grid=(N // bm,),
        in_specs=[pl.BlockSpec((bm, 256), lambda i: (i, 0)),
                  pl.BlockSpec((1, 256), lambda i: (0, 0)),
                  pl.BlockSpec((bm, 256), lambda i: (i, 0)),
                  pl.BlockSpec((bm, 256), lambda i: (i, 0)),
                  pl.BlockSpec((bm, 256), lambda i: (i, 0)),
                  pl.BlockSpec((1, 256), lambda i: (0, 0))],
        out_specs=pl.BlockSpec((bm, 512), lambda i: (i, 0)),
        out_shape=jax.ShapeDtypeStruct((N, 512), jnp.float32),
    )(x1raw, b_lin1.reshape(1, 256), p0, p1, invb2, b2.reshape(1, 256))


def kernel(d_sim, m_sim, diseases, mirnas, W_d, W_m, W_lin1, b_lin1, W1, a_s1, a_d1, b1, W2, a_s2, a_d2, b2, Wp, bp, ln_g, ln_b, Wl, bl, Wd1, bd1, Wm1, bm1, Wpred, bpred):
    z_d = _matmul(d_sim, W_d)
    z_m = _matmul(m_sim, W_m)
    feats = jnp.concatenate([z_d, z_m], axis=0)
    src = diseases.astype(jnp.int32)
    dst = mirnas.astype(jnp.int32)

    x1raw = _matmul(feats, W_lin1)

    eye8 = jnp.eye(8, dtype=jnp.float32)
    A_s1 = (eye8[:, None, :] * a_s1[:, :, None]).reshape(2048, 8)
    A_d1 = (eye8[:, None, :] * a_d1[:, :, None]).reshape(2048, 8)
    A_sd1 = jnp.pad(jnp.concatenate([A_s1, A_d1], axis=1), ((0, 0), (0, 112)))
    W1A = _matmul(W1, A_sd1, bm=64)
    esd1 = _matmul(feats, W1A)
    ES1 = jnp.pad(jnp.concatenate([esd1[:, :8], esd1[:, :8]], axis=1),
                  ((0, 0), (0, 112)))
    ED1 = jnp.pad(jnp.concatenate([esd1[:, 8:16], esd1[:, 8:16]], axis=1),
                  ((0, 0), (0, 112)))
    h1c = _matmul_chunks(feats, W1, 8)
    p0, p1g, inv1 = _gat_sc(h1c, ES1, ED1, src, dst)
    invb1 = jnp.repeat(inv1[:, :8], 256, axis=1)

    h2 = _relu_add_mm(p0, p1g, invb1, b1, W2)
    A2 = jnp.pad(jnp.concatenate([a_s2.T, a_d2.T], axis=1),
                 ((0, 0), (0, 126)))
    esd2 = _matmul(h2, A2)
    ES2 = jnp.pad(jnp.broadcast_to(esd2[:, 0:1], (N, 16)),
                  ((0, 0), (0, 112)))
    ED2 = jnp.pad(jnp.broadcast_to(esd2[:, 1:2], (N, 16)),
                  ((0, 0), (0, 112)))
    q0, q1, inv2 = _gat_sc(h2.reshape(1, N, 256), ES2, ED2, src, dst)
    invb2 = jnp.broadcast_to(inv2[:, 0:1], (N, 256))

    final = _final_assemble(x1raw, b_lin1, q0, q1, invb2, b2)

    Wp_pad = jnp.pad(Wp, ((0, 0), (0, 124)))
    bp_pad = jnp.pad(bp, (0, 124))
    g = _matmul(final, Wp_pad) + bp_pad
    key, v50, tie = _knn_select(g)
    outf = _knn_mix(key, v50, tie, final, ln_g, ln_b, Wl, bl)
    h_d = jnp.concatenate([outf[:ND], feats[:ND]], axis=1)
    h_m = jnp.concatenate([outf[ND:], feats[ND:]], axis=1)
    h_d = jax.nn.elu(_matmul(h_d, Wd1) + bd1)
    h_m = jax.nn.elu(_matmul(h_m, Wm1) + bm1)
    h = jnp.concatenate([h_d, h_m], axis=0)
    h_pad = jnp.pad(h, ((0, 0), (0, 64)))
    hd_rows, hm_rows = _sc_edge_gather(h_pad, src.reshape(_NW, 8, 128),
                                       dst.reshape(_NW, 8, 128))
    hc2 = jnp.concatenate([hd_rows, hm_rows], axis=1)
    Wp_a = jnp.pad(Wpred[:64], ((0, 64), (0, 127)))
    Wp_b = jnp.pad(Wpred[64:], ((0, 64), (0, 127)))
    Wpred2 = jnp.concatenate([Wp_a, Wp_b], axis=0)
    logits = _matmul(hc2, Wpred2)[:, :1]
    return jax.nn.sigmoid(logits + bpred)
